# Initial kernel scaffold; baseline (speedup 1.0000x reference)
import jax
import jax.numpy as jnp
from jax.experimental import pallas as pl


def kernel(X, edge_index, W1, B1, W2, B2):
    raise NotImplementedError("write your pallas kernel here")


# speedup vs baseline: 7.0807x; 7.0807x over previous
import functools

import jax
import jax.numpy as jnp
from jax import lax
from jax.experimental import pallas as pl
from jax.experimental.pallas import tpu as pltpu
from jax.experimental.pallas import tpu_sc as plsc

N_NODES = 10000
N_EDGES = 320000
NC = 2
NS = 16
NW = NC * NS
E_PER_W = N_EDGES // NW
CHUNK = 80
N_CHUNKS = E_PER_W // CHUNK
ROW_OFF = 624
ROW_SPAN = 640


def _make_spmm(F):
  mesh = plsc.VectorSubcoreMesh(core_axis_name="c", subcore_axis_name="s")

  @functools.partial(
      pl.kernel,
      out_type=jax.ShapeDtypeStruct((NC, N_NODES, F), jnp.float32),
      mesh=mesh,
      scratch_types=[
          pltpu.VMEM((N_CHUNKS, CHUNK), jnp.int32),
          pltpu.VMEM((N_CHUNKS, CHUNK), jnp.int32),
          pltpu.VMEM((CHUNK, F), jnp.float32),
          pltpu.VMEM_SHARED((N_NODES, F), jnp.float32),
          pltpu.SemaphoreType.DMA,
      ],
  )
  def spmm(table, src, dst, zeros, out, src_v, dst_v, rows_v, acc, sem):
    c = lax.axis_index("c")
    s = lax.axis_index("s")
    wid = c * NS + s
    row0 = pl.multiple_of(s * ROW_OFF, 8)
    pltpu.sync_copy(src.at[wid], src_v)
    pltpu.sync_copy(dst.at[wid], dst_v)
    pltpu.sync_copy(zeros, acc.at[pl.ds(row0, ROW_SPAN)])
    plsc.subcore_barrier()

    def body(j, carry):
      pltpu.async_copy(table.at[src_v.at[j]], rows_v, sem).wait()
      pltpu.sync_copy(rows_v, acc.at[dst_v.at[j]], add=True)
      return carry

    lax.fori_loop(0, N_CHUNKS, body, 0)
    plsc.subcore_barrier()
    pltpu.sync_copy(acc.at[pl.ds(row0, ROW_SPAN)],
                    out.at[c, pl.ds(row0, ROW_SPAN)])

  return spmm


_spmm_128 = _make_spmm(128)


def _tc1_body(x_ref, w_ref, out_ref):
  out_ref[...] = jnp.dot(x_ref[...], w_ref[...],
                         preferred_element_type=jnp.float32)


def _tc2_body(p_ref, b_ref, out_ref):
  out_ref[...] = jnp.maximum(p_ref[0] + p_ref[1] + b_ref[...], 0.0)


def _tc3_body(p_ref, w_ref, b_ref, out_ref):
  o = jnp.dot(p_ref[0] + p_ref[1], w_ref[...],
              preferred_element_type=jnp.float32) + b_ref[...]
  m = jnp.max(o, axis=1, keepdims=True)
  x = o - m
  lse = jnp.log(jnp.sum(jnp.exp(x), axis=1, keepdims=True))
  out_ref[...] = x - lse


def kernel(X, edge_index, W1, B1, W2, B2):
  src = edge_index[0].astype(jnp.int32).reshape(NW, N_CHUNKS, CHUNK)
  dst = edge_index[1].astype(jnp.int32).reshape(NW, N_CHUNKS, CHUNK)

  s1 = pl.pallas_call(
      _tc1_body,
      out_shape=jax.ShapeDtypeStruct((N_NODES, 128), jnp.float32),
  )(X, W1)

  zeros = jnp.zeros((ROW_SPAN, 128), jnp.float32)
  p1 = _spmm_128(s1, src, dst, zeros)

  h = pl.pallas_call(
      _tc2_body,
      out_shape=jax.ShapeDtypeStruct((N_NODES, 128), jnp.float32),
  )(p1, B1.reshape(1, 128))

  p2 = _spmm_128(h, src, dst, zeros)

  return pl.pallas_call(
      _tc3_body,
      out_shape=jax.ShapeDtypeStruct((N_NODES, 64), jnp.float32),
  )(p2, W2, B2.reshape(1, 64))
